# baseline (speedup 1.0000x reference)
import functools

import jax
import jax.numpy as jnp
from jax import lax
from jax.experimental import pallas as pl
from jax.experimental.pallas import tpu as pltpu
from jax.experimental.pallas import tpu_sc as plsc

N = 10000
E = 320000
D = 128
NL = 8
NG = 1024
NSEG = 16
NCAT = D * (NL - 1)

NCORES = 2
NSUB = 16
CHUNK = 64
CH = 160
NBUF = 4
NIDX = 8
E_PAD = NCORES * NSUB * CH * CHUNK
NACC = 10112
RPW = NACC // NSUB

_mesh = lambda: plsc.VectorSubcoreMesh(core_axis_name="c", subcore_axis_name="s")


def _sc_spmm(h, src2d, dst2d, zeros128):
    @functools.partial(
        pl.kernel,
        out_type=jax.ShapeDtypeStruct((NCORES, NACC, D), jnp.float32),
        mesh=_mesh(),
        scratch_types=(
            [pltpu.VMEM((CHUNK,), jnp.int32) for _ in range(2 * NIDX)]
            + [pltpu.VMEM((CHUNK, D), jnp.float32) for _ in range(NBUF)]
            + [pltpu.SemaphoreType.DMA for _ in range(NIDX + 2 * NBUF)]
            + [pltpu.VMEM_SHARED((NACC, D), jnp.float32)]
        ),
    )
    def k(h_hbm, src_hbm, dst_hbm, z_hbm, out_hbm, *refs):
        sidx = refs[0:NIDX]
        didx = refs[NIDX:2 * NIDX]
        rows = refs[2 * NIDX:2 * NIDX + NBUF]
        semi = refs[2 * NIDX + NBUF:3 * NIDX + NBUF]
        semg = refs[3 * NIDX + NBUF:3 * NIDX + 2 * NBUF]
        semsc = refs[3 * NIDX + 2 * NBUF:3 * NIDX + 3 * NBUF]
        acc = refs[-1]
        c = lax.axis_index("c")
        s = lax.axis_index("s")
        pltpu.sync_copy(z_hbm.at[pl.ds(s * RPW, RPW)], acc.at[pl.ds(s * RPW, RPW)])
        plsc.subcore_barrier()

        def fire_idx(j, sl):
            pltpu.async_copy(src_hbm.at[c, s, j], sidx[sl], semi[sl])
            pltpu.async_copy(dst_hbm.at[c, s, j], didx[sl], semi[sl])

        def wait_idx(sl):
            pltpu.make_async_copy(src_hbm.at[c, s, 0], sidx[sl], semi[sl]).wait()
            pltpu.make_async_copy(dst_hbm.at[c, s, 0], didx[sl], semi[sl]).wait()

        def fire_gather(b, sl):
            pltpu.async_copy(h_hbm.at[sidx[sl]], rows[b], semg[b])

        def wait_gather(b, sl):
            pltpu.make_async_copy(h_hbm.at[sidx[sl]], rows[b], semg[b]).wait()

        def fire_scatter(b, sl):
            pltpu.async_copy(rows[b], acc.at[didx[sl]], semsc[b], add=True)

        def drain_scatter(b, sl):
            pltpu.make_async_copy(rows[b], acc.at[didx[sl]], semsc[b]).wait()

        for j in range(6):
            fire_idx(j, j)
        for j in range(2):
            wait_idx(j)
            fire_gather(j, j)

        def body(i, carry):
            for u in range(8):
                cc = 8 * i + u
                b, bi = u % NBUF, u % NIDX
                b2, bi2 = (u + 2) % NBUF, (u + 2) % NIDX
                bi6 = (u + 6) % NIDX
                wait_gather(b, bi)
                fire_scatter(b, bi)

                @pl.when((cc >= 2) & (cc + 2 < CH))
                def _():
                    drain_scatter(b2, bi6)

                @pl.when(cc + 2 < CH)
                def _():
                    wait_idx(bi2)
                    fire_gather(b2, bi2)

                @pl.when(cc + 6 < CH)
                def _():
                    fire_idx(cc + 6, bi6)

            return carry

        lax.fori_loop(0, CH // 8, body, 0)
        for t in range(NBUF):
            drain_scatter(t, 4 + t)
        plsc.subcore_barrier()
        pltpu.sync_copy(acc.at[pl.ds(s * RPW, RPW)],
                        out_hbm.at[c, pl.ds(s * RPW, RPW)])

    return k(h, src2d, dst2d, zeros128)


_B = 1000


def _tc_prep(x, degp, interpret=False):
    def body(x_ref, d_ref, o_ref):
        deg = jnp.maximum(d_ref[0, :, :1] + d_ref[1, :, :1], 1.0)
        o_ref[...] = x_ref[...] * lax.rsqrt(deg)

    return pl.pallas_call(
        body,
        grid=(N // _B,),
        in_specs=[
            pl.BlockSpec((_B, D), lambda i: (i, 0)),
            pl.BlockSpec((2, _B, 16), lambda i: (0, i, 0)),
        ],
        out_specs=pl.BlockSpec((_B, D), lambda i: (i, 0)),
        out_shape=jax.ShapeDtypeStruct((N, D), jnp.float32),
        interpret=interpret,
    )(x, degp)


def _tc_conv(p, degp, h_prev, W, b, has_res, interpret=False):
    def body(p_ref, d_ref, hp_ref, w_ref, b_ref, oh_ref, os_ref):
        deg = jnp.maximum(d_ref[0, :, :1] + d_ref[1, :, :1], 1.0)
        dinv = lax.rsqrt(deg)
        agg = (p_ref[0] + p_ref[1]) * dinv
        h = jnp.dot(agg, w_ref[...], preferred_element_type=jnp.float32)
        h = jnp.maximum(h + b_ref[...], 0.0)
        if has_res:
            h = h + hp_ref[...]
        oh_ref[...] = h
        os_ref[...] = h * dinv

    return pl.pallas_call(
        body,
        grid=(N // _B,),
        in_specs=[
            pl.BlockSpec((2, _B, D), lambda i: (0, i, 0)),
            pl.BlockSpec((2, _B, 16), lambda i: (0, i, 0)),
            pl.BlockSpec((_B, D), lambda i: (i, 0)),
            pl.BlockSpec((D, D), lambda i: (0, 0)),
            pl.BlockSpec((1, D), lambda i: (0, 0)),
        ],
        out_specs=[
            pl.BlockSpec((_B, D), lambda i: (i, 0)),
            pl.BlockSpec((_B, D), lambda i: (i, 0)),
        ],
        out_shape=[
            jax.ShapeDtypeStruct((N, D), jnp.float32),
            jax.ShapeDtypeStruct((N, D), jnp.float32),
        ],
        interpret=interpret,
    )(p, degp, h_prev, W, b)


def _tc_xgpool(xc, onehot, Wc, bc, interpret=False):
    def body(xc_ref, oh_ref, w_ref, b_ref, ssum_ref, ssq_ref, smax_ref, cnt_ref):
        @pl.when(pl.program_id(0) == 0)
        def _():
            ssum_ref[...] = jnp.zeros_like(ssum_ref)
            ssq_ref[...] = jnp.zeros_like(ssq_ref)
            smax_ref[...] = jnp.zeros_like(smax_ref)
            cnt_ref[...] = jnp.zeros_like(cnt_ref)

        xg = jnp.dot(xc_ref[...], w_ref[...], preferred_element_type=jnp.float32)
        xg = jnp.maximum(xg + b_ref[...], 0.0)
        oh = oh_ref[...]
        dn = (((0,), (0,)), ((), ()))
        ssum_ref[...] += lax.dot_general(oh, xg, dn,
                                         preferred_element_type=jnp.float32)
        ssq_ref[...] += lax.dot_general(oh, xg * xg, dn,
                                        preferred_element_type=jnp.float32)
        cnt_ref[...] += jnp.sum(oh, axis=0)[:, None] * jnp.ones((1, 128), jnp.float32)
        mx = jnp.stack([jnp.max(oh[:, s_:s_ + 1] * xg, axis=0)
                        for s_ in range(NSEG)])
        smax_ref[...] = jnp.maximum(smax_ref[...], mx)

    return pl.pallas_call(
        body,
        grid=(N // _B,),
        in_specs=[
            pl.BlockSpec((_B, NCAT), lambda i: (i, 0)),
            pl.BlockSpec((_B, NSEG), lambda i: (i, 0)),
            pl.BlockSpec((NCAT, NG), lambda i: (0, 0)),
            pl.BlockSpec((1, NG), lambda i: (0, 0)),
        ],
        out_specs=[
            pl.BlockSpec((NSEG, NG), lambda i: (0, 0)),
            pl.BlockSpec((NSEG, NG), lambda i: (0, 0)),
            pl.BlockSpec((NSEG, NG), lambda i: (0, 0)),
            pl.BlockSpec((NSEG, 128), lambda i: (0, 0)),
        ],
        out_shape=[
            jax.ShapeDtypeStruct((NSEG, NG), jnp.float32),
            jax.ShapeDtypeStruct((NSEG, NG), jnp.float32),
            jax.ShapeDtypeStruct((NSEG, NG), jnp.float32),
            jax.ShapeDtypeStruct((NSEG, 128), jnp.float32),
        ],
        interpret=interpret,
    )(xc, onehot, Wc, bc)


def _tc_head(ssum, ssq, smax, cnt, Wmax, Wmean, Wstd, bt1, interpret=False):
    def body(ssum_ref, ssq_ref, smax_ref, cnt_ref, wm_ref, wu_ref, ws_ref,
             b_ref, o_ref):
        cnt = cnt_ref[:, :1]
        cntc = jnp.maximum(cnt, 1.0)
        mean = ssum_ref[...] / cntc
        ss = ssq_ref[...] - 2.0 * mean * ssum_ref[...] + cnt * mean * mean
        ss = jnp.maximum(ss, 0.0)
        std = jnp.sqrt(ss / jnp.maximum(cnt - 1.0, 1.0))
        acc = jnp.dot(smax_ref[...], wm_ref[...],
                      preferred_element_type=jnp.float32)
        acc += jnp.dot(mean, wu_ref[...], preferred_element_type=jnp.float32)
        acc += jnp.dot(std, ws_ref[...], preferred_element_type=jnp.float32)
        o_ref[...] = acc + b_ref[...]

    return pl.pallas_call(
        body,
        out_shape=jax.ShapeDtypeStruct((NSEG, 2048), jnp.float32),
        interpret=interpret,
    )(ssum, ssq, smax, cnt, Wmax, Wmean, Wstd, bt1)


def _ln_relu(z, g, b):
    m = jnp.mean(z, axis=-1, keepdims=True)
    d = z - m
    v = jnp.mean(d * d, axis=-1, keepdims=True)
    return jnp.maximum(d * lax.rsqrt(v + 1e-5) * g + b, 0.0)


def _tc_mlp(xc, onehot, pooled_b, Wt1a, g1, be1, Wt2, bt2, g2, be2, Wt3p, bt3p,
            interpret=False):
    def body(xc_ref, oh_ref, pb_ref, w1_ref, g1_ref, b1_ref, w2_ref, bt2_ref,
             g2_ref, b2_ref, w3_ref, bt3_ref, o_ref):
        z1 = jnp.dot(xc_ref[...], w1_ref[...], preferred_element_type=jnp.float32)
        z1 += jnp.dot(oh_ref[...], pb_ref[...], preferred_element_type=jnp.float32)
        a1 = _ln_relu(z1, g1_ref[...], b1_ref[...])
        z2 = jnp.dot(a1, w2_ref[...], preferred_element_type=jnp.float32)
        a2 = _ln_relu(z2 + bt2_ref[...], g2_ref[...], b2_ref[...])
        o_ref[...] = jnp.dot(a2, w3_ref[...],
                             preferred_element_type=jnp.float32) + bt3_ref[...]

    return pl.pallas_call(
        body,
        grid=(N // _B,),
        in_specs=[
            pl.BlockSpec((_B, NCAT), lambda i: (i, 0)),
            pl.BlockSpec((_B, NSEG), lambda i: (i, 0)),
            pl.BlockSpec((NSEG, 2048), lambda i: (0, 0)),
            pl.BlockSpec((NCAT, 2048), lambda i: (0, 0)),
            pl.BlockSpec((1, 2048), lambda i: (0, 0)),
            pl.BlockSpec((1, 2048), lambda i: (0, 0)),
            pl.BlockSpec((2048, 2048), lambda i: (0, 0)),
            pl.BlockSpec((1, 2048), lambda i: (0, 0)),
            pl.BlockSpec((1, 2048), lambda i: (0, 0)),
            pl.BlockSpec((1, 2048), lambda i: (0, 0)),
            pl.BlockSpec((2048, 128), lambda i: (0, 0)),
            pl.BlockSpec((1, 128), lambda i: (0, 0)),
        ],
        out_specs=pl.BlockSpec((_B, 128), lambda i: (i, 0)),
        out_shape=jax.ShapeDtypeStruct((N, 128), jnp.float32),
        interpret=interpret,
    )(xc, onehot, pooled_b, Wt1a, g1, be1, Wt2, bt2, g2, be2, Wt3p, bt3p)


def kernel(x, edge_indices, batch, W0, b0, Ws, bs, Wc, bc, Wt1, bt1, g1, be1,
           Wt2, bt2, g2, be2, Wt3, bt3):
    src = edge_indices[0]
    dst = edge_indices[1]
    pad = E_PAD - E
    src_p = jnp.concatenate([src, jnp.zeros((pad,), jnp.int32)])
    dst_p = jnp.concatenate([dst, jnp.full((pad,), N, jnp.int32)])
    src2d = src_p.reshape(NCORES, NSUB, CH, CHUNK)
    dst2d = dst_p.reshape(NCORES, NSUB, CH, CHUNK)
    zeros128 = jnp.zeros((NACC, D), jnp.float32)

    degp = _sc_spmm(jnp.ones((N, D), jnp.float32), jnp.zeros_like(src2d),
                    dst2d, zeros128)[:, :, :16]

    hs = _tc_prep(x, degp)
    h = jnp.zeros((N, D), jnp.float32)
    outs = []
    for l in range(NL):
        p = _sc_spmm(hs, src2d, dst2d, zeros128)
        W = W0 if l == 0 else Ws[l - 1]
        b = (b0 if l == 0 else bs[l - 1]).reshape(1, D)
        h, hs = _tc_conv(p, degp, h, W, b, has_res=(l > 0))
        if l > 0:
            outs.append(h)
    xc = jnp.concatenate(outs, axis=1)

    onehot = (batch[:, None] == jnp.arange(NSEG, dtype=batch.dtype)
              ).astype(jnp.float32)
    ssum, ssq, smax, cnt = _tc_xgpool(xc, onehot, Wc, bc.reshape(1, NG))
    pooled_b = _tc_head(ssum, ssq, smax, cnt,
                        Wt1[NCAT:NCAT + NG],
                        Wt1[NCAT + NG:NCAT + 2 * NG],
                        Wt1[NCAT + 2 * NG:],
                        bt1.reshape(1, 2048))
    Wt3p = jnp.pad(Wt3, ((0, 0), (0, 127)))
    bt3p = jnp.pad(bt3, (0, 127)).reshape(1, 128)
    res = _tc_mlp(xc, onehot, pooled_b, Wt1[:NCAT], g1.reshape(1, 2048),
                  be1.reshape(1, 2048), Wt2, bt2.reshape(1, 2048),
                  g2.reshape(1, 2048), be2.reshape(1, 2048), Wt3p, bt3p)
    return res[:, :1]


# speedup vs baseline: 4.4857x; 4.4857x over previous
import functools

import jax
import jax.numpy as jnp
from jax import lax
from jax.experimental import pallas as pl
from jax.experimental.pallas import tpu as pltpu
from jax.experimental.pallas import tpu_sc as plsc

N = 10000
E = 320000
D = 128
NL = 8
NG = 1024
NSEG = 16
NCAT = D * (NL - 1)

NCORES = 2
NSUB = 16
CHUNK = 64
CH = 160
NBUF = 4
NIDX = 8
E_PAD = NCORES * NSUB * CH * CHUNK
NACC = 10112
RPW = NACC // NSUB

_mesh = lambda: plsc.VectorSubcoreMesh(core_axis_name="c", subcore_axis_name="s")


def _sc_spmm(h, src2d, dst2d, zeros128, const_rows=False):
    @functools.partial(
        pl.kernel,
        out_type=jax.ShapeDtypeStruct((NCORES, NACC, D), jnp.float32),
        mesh=_mesh(),
        scratch_types=(
            [pltpu.VMEM((CHUNK,), jnp.int32) for _ in range(2 * NIDX)]
            + [pltpu.VMEM((CHUNK, D), jnp.float32) for _ in range(NBUF)]
            + [pltpu.SemaphoreType.DMA for _ in range(NIDX + 2 * NBUF)]
            + [pltpu.VMEM_SHARED((NACC, D), jnp.float32)]
        ),
    )
    def k(h_hbm, src_hbm, dst_hbm, z_hbm, out_hbm, *refs):
        sidx = refs[0:NIDX]
        didx = refs[NIDX:2 * NIDX]
        rows = refs[2 * NIDX:2 * NIDX + NBUF]
        semi = refs[2 * NIDX + NBUF:3 * NIDX + NBUF]
        semg = refs[3 * NIDX + NBUF:3 * NIDX + 2 * NBUF]
        semsc = refs[3 * NIDX + 2 * NBUF:3 * NIDX + 3 * NBUF]
        acc = refs[-1]
        c = lax.axis_index("c")
        s = lax.axis_index("s")
        pltpu.sync_copy(z_hbm.at[pl.ds(s * RPW, RPW)], acc.at[pl.ds(s * RPW, RPW)])
        plsc.subcore_barrier()

        def fire_idx(j, sl):
            if not const_rows:
                pltpu.async_copy(src_hbm.at[c, s, j], sidx[sl], semi[sl])
            pltpu.async_copy(dst_hbm.at[c, s, j], didx[sl], semi[sl])

        def wait_idx(sl):
            if not const_rows:
                pltpu.make_async_copy(src_hbm.at[c, s, 0], sidx[sl],
                                      semi[sl]).wait()
            pltpu.make_async_copy(dst_hbm.at[c, s, 0], didx[sl], semi[sl]).wait()

        def fire_gather(b, sl):
            pltpu.async_copy(h_hbm.at[sidx[sl]], rows[b], semg[b])

        def wait_gather(b, sl):
            pltpu.make_async_copy(h_hbm.at[sidx[sl]], rows[b], semg[b]).wait()

        def fire_scatter(b, sl):
            pltpu.async_copy(rows[0] if const_rows else rows[b],
                             acc.at[didx[sl]], semsc[b], add=True)

        def drain_scatter(b, sl):
            pltpu.make_async_copy(rows[0] if const_rows else rows[b],
                                  acc.at[didx[sl]], semsc[b]).wait()

        if const_rows:
            pltpu.sync_copy(h_hbm.at[pl.ds(0, CHUNK)], rows[0])
        for j in range(6):
            fire_idx(j, j)
        if not const_rows:
            for j in range(2):
                wait_idx(j)
                fire_gather(j, j)

        def body(i, carry):
            for u in range(8):
                cc = 8 * i + u
                b, bi = u % NBUF, u % NIDX
                b2, bi2 = (u + 2) % NBUF, (u + 2) % NIDX
                bi6 = (u + 6) % NIDX
                if const_rows:
                    wait_idx(bi)
                else:
                    wait_gather(b, bi)
                fire_scatter(b, bi)

                @pl.when((cc >= 2) & (cc + 2 < CH))
                def _():
                    drain_scatter(b2, bi6)

                if not const_rows:
                    @pl.when(cc + 2 < CH)
                    def _():
                        wait_idx(bi2)
                        fire_gather(b2, bi2)

                @pl.when(cc + 6 < CH)
                def _():
                    fire_idx(cc + 6, bi6)

            return carry

        lax.fori_loop(0, CH // 8, body, 0)
        for t in range(NBUF):
            drain_scatter(t, 4 + t)
        plsc.subcore_barrier()
        pltpu.sync_copy(acc.at[pl.ds(s * RPW, RPW)],
                        out_hbm.at[c, pl.ds(s * RPW, RPW)])

    return k(h, src2d, dst2d, zeros128)


_B = 1000


def _tc_prep(x, degp, interpret=False):
    def body(x_ref, d_ref, o_ref):
        deg = jnp.maximum(d_ref[0, :, :1] + d_ref[1, :, :1], 1.0)
        o_ref[...] = x_ref[...] * lax.rsqrt(deg)

    return pl.pallas_call(
        body,
        grid=(N // _B,),
        in_specs=[
            pl.BlockSpec((_B, D), lambda i: (i, 0)),
            pl.BlockSpec((2, _B, 16), lambda i: (0, i, 0)),
        ],
        out_specs=pl.BlockSpec((_B, D), lambda i: (i, 0)),
        out_shape=jax.ShapeDtypeStruct((N, D), jnp.float32),
        interpret=interpret,
    )(x, degp)


def _tc_conv(p, degp, h_prev, W, b, has_res, interpret=False):
    def body(p_ref, d_ref, hp_ref, w_ref, b_ref, oh_ref, os_ref):
        deg = jnp.maximum(d_ref[0, :, :1] + d_ref[1, :, :1], 1.0)
        dinv = lax.rsqrt(deg)
        agg = (p_ref[0] + p_ref[1]) * dinv
        h = jnp.dot(agg, w_ref[...], preferred_element_type=jnp.float32)
        h = jnp.maximum(h + b_ref[...], 0.0)
        if has_res:
            h = h + hp_ref[...]
        oh_ref[...] = h
        os_ref[...] = h * dinv

    return pl.pallas_call(
        body,
        grid=(N // _B,),
        in_specs=[
            pl.BlockSpec((2, _B, D), lambda i: (0, i, 0)),
            pl.BlockSpec((2, _B, 16), lambda i: (0, i, 0)),
            pl.BlockSpec((_B, D), lambda i: (i, 0)),
            pl.BlockSpec((D, D), lambda i: (0, 0)),
            pl.BlockSpec((1, D), lambda i: (0, 0)),
        ],
        out_specs=[
            pl.BlockSpec((_B, D), lambda i: (i, 0)),
            pl.BlockSpec((_B, D), lambda i: (i, 0)),
        ],
        out_shape=[
            jax.ShapeDtypeStruct((N, D), jnp.float32),
            jax.ShapeDtypeStruct((N, D), jnp.float32),
        ],
        interpret=interpret,
    )(p, degp, h_prev, W, b)


def _tc_xgpool(xc, onehot, Wc, bc, interpret=False):
    def body(xc_ref, oh_ref, w_ref, b_ref, ssum_ref, ssq_ref, smax_ref, cnt_ref):
        @pl.when(pl.program_id(0) == 0)
        def _():
            ssum_ref[...] = jnp.zeros_like(ssum_ref)
            ssq_ref[...] = jnp.zeros_like(ssq_ref)
            smax_ref[...] = jnp.zeros_like(smax_ref)
            cnt_ref[...] = jnp.zeros_like(cnt_ref)

        xg = jnp.dot(xc_ref[...], w_ref[...], preferred_element_type=jnp.float32)
        xg = jnp.maximum(xg + b_ref[...], 0.0)
        oh = oh_ref[...]
        dn = (((0,), (0,)), ((), ()))
        ssum_ref[...] += lax.dot_general(oh, xg, dn,
                                         preferred_element_type=jnp.float32)
        ssq_ref[...] += lax.dot_general(oh, xg * xg, dn,
                                        preferred_element_type=jnp.float32)
        cnt_ref[...] += jnp.sum(oh, axis=0)[:, None] * jnp.ones((1, 128), jnp.float32)
        mx = jnp.stack([jnp.max(oh[:, s_:s_ + 1] * xg, axis=0)
                        for s_ in range(NSEG)])
        smax_ref[...] = jnp.maximum(smax_ref[...], mx)

    return pl.pallas_call(
        body,
        grid=(N // _B,),
        in_specs=[
            pl.BlockSpec((_B, NCAT), lambda i: (i, 0)),
            pl.BlockSpec((_B, NSEG), lambda i: (i, 0)),
            pl.BlockSpec((NCAT, NG), lambda i: (0, 0)),
            pl.BlockSpec((1, NG), lambda i: (0, 0)),
        ],
        out_specs=[
            pl.BlockSpec((NSEG, NG), lambda i: (0, 0)),
            pl.BlockSpec((NSEG, NG), lambda i: (0, 0)),
            pl.BlockSpec((NSEG, NG), lambda i: (0, 0)),
            pl.BlockSpec((NSEG, 128), lambda i: (0, 0)),
        ],
        out_shape=[
            jax.ShapeDtypeStruct((NSEG, NG), jnp.float32),
            jax.ShapeDtypeStruct((NSEG, NG), jnp.float32),
            jax.ShapeDtypeStruct((NSEG, NG), jnp.float32),
            jax.ShapeDtypeStruct((NSEG, 128), jnp.float32),
        ],
        interpret=interpret,
    )(xc, onehot, Wc, bc)


def _tc_head(ssum, ssq, smax, cnt, Wmax, Wmean, Wstd, bt1, interpret=False):
    def body(ssum_ref, ssq_ref, smax_ref, cnt_ref, wm_ref, wu_ref, ws_ref,
             b_ref, o_ref):
        cnt = cnt_ref[:, :1]
        cntc = jnp.maximum(cnt, 1.0)
        mean = ssum_ref[...] / cntc
        ss = ssq_ref[...] - 2.0 * mean * ssum_ref[...] + cnt * mean * mean
        ss = jnp.maximum(ss, 0.0)
        std = jnp.sqrt(ss / jnp.maximum(cnt - 1.0, 1.0))
        acc = jnp.dot(smax_ref[...], wm_ref[...],
                      preferred_element_type=jnp.float32)
        acc += jnp.dot(mean, wu_ref[...], preferred_element_type=jnp.float32)
        acc += jnp.dot(std, ws_ref[...], preferred_element_type=jnp.float32)
        o_ref[...] = acc + b_ref[...]

    return pl.pallas_call(
        body,
        out_shape=jax.ShapeDtypeStruct((NSEG, 2048), jnp.float32),
        interpret=interpret,
    )(ssum, ssq, smax, cnt, Wmax, Wmean, Wstd, bt1)


def _ln_relu(z, g, b):
    m = jnp.mean(z, axis=-1, keepdims=True)
    d = z - m
    v = jnp.mean(d * d, axis=-1, keepdims=True)
    return jnp.maximum(d * lax.rsqrt(v + 1e-5) * g + b, 0.0)


def _tc_mlp(xc, onehot, pooled_b, Wt1a, g1, be1, Wt2, bt2, g2, be2, Wt3p, bt3p,
            interpret=False):
    def body(xc_ref, oh_ref, pb_ref, w1_ref, g1_ref, b1_ref, w2_ref, bt2_ref,
             g2_ref, b2_ref, w3_ref, bt3_ref, o_ref):
        z1 = jnp.dot(xc_ref[...], w1_ref[...], preferred_element_type=jnp.float32)
        z1 += jnp.dot(oh_ref[...], pb_ref[...], preferred_element_type=jnp.float32)
        a1 = _ln_relu(z1, g1_ref[...], b1_ref[...])
        z2 = jnp.dot(a1, w2_ref[...], preferred_element_type=jnp.float32)
        a2 = _ln_relu(z2 + bt2_ref[...], g2_ref[...], b2_ref[...])
        o_ref[...] = jnp.dot(a2, w3_ref[...],
                             preferred_element_type=jnp.float32) + bt3_ref[...]

    return pl.pallas_call(
        body,
        grid=(N // _B,),
        in_specs=[
            pl.BlockSpec((_B, NCAT), lambda i: (i, 0)),
            pl.BlockSpec((_B, NSEG), lambda i: (i, 0)),
            pl.BlockSpec((NSEG, 2048), lambda i: (0, 0)),
            pl.BlockSpec((NCAT, 2048), lambda i: (0, 0)),
            pl.BlockSpec((1, 2048), lambda i: (0, 0)),
            pl.BlockSpec((1, 2048), lambda i: (0, 0)),
            pl.BlockSpec((2048, 2048), lambda i: (0, 0)),
            pl.BlockSpec((1, 2048), lambda i: (0, 0)),
            pl.BlockSpec((1, 2048), lambda i: (0, 0)),
            pl.BlockSpec((1, 2048), lambda i: (0, 0)),
            pl.BlockSpec((2048, 128), lambda i: (0, 0)),
            pl.BlockSpec((1, 128), lambda i: (0, 0)),
        ],
        out_specs=pl.BlockSpec((_B, 128), lambda i: (i, 0)),
        out_shape=jax.ShapeDtypeStruct((N, 128), jnp.float32),
        interpret=interpret,
    )(xc, onehot, pooled_b, Wt1a, g1, be1, Wt2, bt2, g2, be2, Wt3p, bt3p)


def kernel(x, edge_indices, batch, W0, b0, Ws, bs, Wc, bc, Wt1, bt1, g1, be1,
           Wt2, bt2, g2, be2, Wt3, bt3):
    src = edge_indices[0]
    dst = edge_indices[1]
    pad = E_PAD - E
    src_p = jnp.concatenate([src, jnp.zeros((pad,), jnp.int32)])
    dst_p = jnp.concatenate([dst, jnp.full((pad,), N, jnp.int32)])
    src2d = src_p.reshape(NCORES, NSUB, CH, CHUNK)
    dst2d = dst_p.reshape(NCORES, NSUB, CH, CHUNK)
    zeros128 = jnp.zeros((NACC, D), jnp.float32)

    degp = _sc_spmm(jnp.ones((CHUNK, D), jnp.float32), src2d, dst2d,
                    zeros128, const_rows=True)[:, :, :16]

    hs = _tc_prep(x, degp)
    h = jnp.zeros((N, D), jnp.float32)
    outs = []
    for l in range(NL):
        p = _sc_spmm(hs, src2d, dst2d, zeros128)
        W = W0 if l == 0 else Ws[l - 1]
        b = (b0 if l == 0 else bs[l - 1]).reshape(1, D)
        h, hs = _tc_conv(p, degp, h, W, b, has_res=(l > 0))
        if l > 0:
            outs.append(h)
    xc = jnp.concatenate(outs, axis=1)

    onehot = (batch[:, None] == jnp.arange(NSEG, dtype=batch.dtype)
              ).astype(jnp.float32)
    ssum, ssq, smax, cnt = _tc_xgpool(xc, onehot, Wc, bc.reshape(1, NG))
    pooled_b = _tc_head(ssum, ssq, smax, cnt,
                        Wt1[NCAT:NCAT + NG],
                        Wt1[NCAT + NG:NCAT + 2 * NG],
                        Wt1[NCAT + 2 * NG:],
                        bt1.reshape(1, 2048))
    Wt3p = jnp.pad(Wt3, ((0, 0), (0, 127)))
    bt3p = jnp.pad(bt3, (0, 127)).reshape(1, 128)
    res = _tc_mlp(xc, onehot, pooled_b, Wt1[:NCAT], g1.reshape(1, 2048),
                  be1.reshape(1, 2048), Wt2, bt2.reshape(1, 2048),
                  g2.reshape(1, 2048), be2.reshape(1, 2048), Wt3p, bt3p)
    return res[:, :1]
